# Initial kernel scaffold; baseline (speedup 1.0000x reference)
import jax
import jax.numpy as jnp
from jax.experimental import pallas as pl


def kernel(flat, cu_seqlens, W1, b1, W2, b2, Wc, bc):
    raise NotImplementedError("write your pallas kernel here")


# speedup vs baseline: 72.1738x; 72.1738x over previous
import functools

import jax
import jax.numpy as jnp
from jax.experimental import pallas as pl
from jax.experimental.pallas import tpu as pltpu

_B = 16
_NH = 8
_D = 128
_H = 64
_NC = 2
_BLK = 2048


def _mil_kernel(starts_ref, ends_ref, flat_ref, w1_ref, b1_ref, w2_ref,
                b2_ref, e_ref, wc_ref, bc_ref, out_ref, acc_ref, accw_ref):
    i = pl.program_id(0)
    nsteps = pl.num_programs(0)

    @pl.when(i == 0)
    def _init():
        acc_ref[...] = jnp.zeros_like(acc_ref)
        accw_ref[...] = jnp.zeros_like(accw_ref)

    x = flat_ref[...]
    h = jnp.tanh(
        jnp.dot(x, w1_ref[...], preferred_element_type=jnp.float32)
        + b1_ref[...])
    att = jax.nn.softplus(
        jnp.dot(h, w2_ref[...], preferred_element_type=jnp.float32)
        + b2_ref[...])
    att_big = jnp.dot(att, e_ref[...], preferred_element_type=jnp.float32)
    row = i * _BLK + jax.lax.broadcasted_iota(jnp.int32, (_BLK, _B * _NH), 0)
    in_bag = jnp.logical_and(row >= starts_ref[...], row < ends_ref[...])
    m = jnp.where(in_bag, att_big, 0.0)
    acc_ref[...] += jax.lax.dot_general(
        m, x, (((0,), (0,)), ((), ())), preferred_element_type=jnp.float32)
    ones = jnp.ones((_BLK, 1), dtype=jnp.float32)
    accw_ref[...] += jax.lax.dot_general(
        m, ones, (((0,), (0,)), ((), ())), preferred_element_type=jnp.float32)

    @pl.when(i == nsteps - 1)
    def _finish():
        wsum = accw_ref[...]
        denom = jnp.where(wsum == 0.0, 1.0, wsum)
        norm = acc_ref[...] / denom
        logits = jnp.zeros((_B, _NC), dtype=jnp.float32) + bc_ref[...]
        for j in range(_NH):
            logits = logits + jnp.dot(
                norm[j * _B:(j + 1) * _B, :],
                wc_ref[j * _D:(j + 1) * _D, :],
                preferred_element_type=jnp.float32)
        out_ref[...] = logits


@jax.jit
def kernel(flat, cu_seqlens, W1, b1, W2, b2, Wc, bc):
    total = flat.shape[0]
    assert total % _BLK == 0
    nsteps = total // _BLK
    starts = jnp.tile(cu_seqlens[:_B], (_NH,)).reshape(1, _B * _NH)
    ends = jnp.tile(cu_seqlens[1:_B + 1], (_NH,)).reshape(1, _B * _NH)
    e = (jax.lax.broadcasted_iota(jnp.int32, (_NH, _B * _NH), 1) // _B
         == jax.lax.broadcasted_iota(jnp.int32, (_NH, _B * _NH), 0)
         ).astype(jnp.float32)

    return pl.pallas_call(
        _mil_kernel,
        grid=(nsteps,),
        in_specs=[
            pl.BlockSpec((1, _B * _NH), lambda i: (0, 0)),
            pl.BlockSpec((1, _B * _NH), lambda i: (0, 0)),
            pl.BlockSpec((_BLK, _D), lambda i: (i, 0)),
            pl.BlockSpec((_D, _H), lambda i: (0, 0)),
            pl.BlockSpec((1, _H), lambda i: (0, 0)),
            pl.BlockSpec((_H, _NH), lambda i: (0, 0)),
            pl.BlockSpec((1, _NH), lambda i: (0, 0)),
            pl.BlockSpec((_NH, _B * _NH), lambda i: (0, 0)),
            pl.BlockSpec((_NH * _D, _NC), lambda i: (0, 0)),
            pl.BlockSpec((1, _NC), lambda i: (0, 0)),
        ],
        out_specs=pl.BlockSpec((_B, _NC), lambda i: (0, 0)),
        out_shape=jax.ShapeDtypeStruct((_B, _NC), jnp.float32),
        scratch_shapes=[
            pltpu.VMEM((_B * _NH, _D), jnp.float32),
            pltpu.VMEM((_B * _NH, 1), jnp.float32),
        ],
        compiler_params=pltpu.CompilerParams(
            dimension_semantics=("arbitrary",)),
    )(starts, ends, flat, W1, b1.reshape(1, _H), W2, b2.reshape(1, _NH),
      e, Wc, bc.reshape(1, _NC))


# speedup vs baseline: 82.0722x; 1.1371x over previous
import functools

import jax
import jax.numpy as jnp
from jax.experimental import pallas as pl
from jax.experimental.pallas import tpu as pltpu

_B = 16
_NH = 8
_D = 128
_H = 64
_NC = 2
_BLK = 4096


def _mil_kernel(starts_ref, ends_ref, flat_ref, w1_ref, b1_ref, w2e_ref,
                b2e_ref, wc_ref, bc_ref, out_ref, acc_ref, accw_ref):
    i = pl.program_id(0)
    nsteps = pl.num_programs(0)

    @pl.when(i == 0)
    def _init():
        acc_ref[...] = jnp.zeros_like(acc_ref)
        accw_ref[...] = jnp.zeros_like(accw_ref)

    x = flat_ref[...]
    h = jnp.tanh(
        jnp.dot(x, w1_ref[...], preferred_element_type=jnp.float32)
        + b1_ref[...])
    att_big = jax.nn.softplus(
        jnp.dot(h, w2e_ref[...], preferred_element_type=jnp.float32)
        + b2e_ref[...])
    row = i * _BLK + jax.lax.broadcasted_iota(jnp.int32, (_BLK, _B * _NH), 0)
    in_bag = jnp.logical_and(row >= starts_ref[...], row < ends_ref[...])
    m = jnp.where(in_bag, att_big, 0.0)
    acc_ref[...] += jax.lax.dot_general(
        m, x, (((0,), (0,)), ((), ())), preferred_element_type=jnp.float32)
    accw_ref[...] += jnp.sum(m, axis=0, keepdims=True)

    @pl.when(i == nsteps - 1)
    def _finish():
        wsum = accw_ref[...]
        denom = jnp.where(wsum == 0.0, 1.0, wsum)
        eye = (jax.lax.broadcasted_iota(jnp.int32, (_B * _NH, _B * _NH), 0)
               == jax.lax.broadcasted_iota(
                   jnp.int32, (_B * _NH, _B * _NH), 1)).astype(jnp.float32)
        denom_col = jax.lax.dot_general(
            eye, denom, (((1,), (1,)), ((), ())),
            preferred_element_type=jnp.float32)
        norm = acc_ref[...] / denom_col
        logits = jnp.zeros((_B, _NC), dtype=jnp.float32) + bc_ref[...]
        for j in range(_NH):
            logits = logits + jnp.dot(
                norm[j * _B:(j + 1) * _B, :],
                wc_ref[j * _D:(j + 1) * _D, :],
                preferred_element_type=jnp.float32)
        out_ref[...] = logits


@jax.jit
def kernel(flat, cu_seqlens, W1, b1, W2, b2, Wc, bc):
    total = flat.shape[0]
    assert total % _BLK == 0
    nsteps = total // _BLK
    starts = jnp.tile(cu_seqlens[:_B], (_NH,)).reshape(1, _B * _NH)
    ends = jnp.tile(cu_seqlens[1:_B + 1], (_NH,)).reshape(1, _B * _NH)
    w2e = jnp.repeat(W2, _B, axis=1)
    b2e = jnp.repeat(b2, _B).reshape(1, _B * _NH)

    return pl.pallas_call(
        _mil_kernel,
        grid=(nsteps,),
        in_specs=[
            pl.BlockSpec((1, _B * _NH), lambda i: (0, 0)),
            pl.BlockSpec((1, _B * _NH), lambda i: (0, 0)),
            pl.BlockSpec((_BLK, _D), lambda i: (i, 0)),
            pl.BlockSpec((_D, _H), lambda i: (0, 0)),
            pl.BlockSpec((1, _H), lambda i: (0, 0)),
            pl.BlockSpec((_H, _B * _NH), lambda i: (0, 0)),
            pl.BlockSpec((1, _B * _NH), lambda i: (0, 0)),
            pl.BlockSpec((_NH * _D, _NC), lambda i: (0, 0)),
            pl.BlockSpec((1, _NC), lambda i: (0, 0)),
        ],
        out_specs=pl.BlockSpec((_B, _NC), lambda i: (0, 0)),
        out_shape=jax.ShapeDtypeStruct((_B, _NC), jnp.float32),
        scratch_shapes=[
            pltpu.VMEM((_B * _NH, _D), jnp.float32),
            pltpu.VMEM((1, _B * _NH), jnp.float32),
        ],
        compiler_params=pltpu.CompilerParams(
            dimension_semantics=("arbitrary",)),
    )(starts, ends, flat, W1, b1.reshape(1, _H), w2e, b2e,
      Wc, bc.reshape(1, _NC))


# speedup vs baseline: 90.7266x; 1.1054x over previous
import functools

import jax
import jax.numpy as jnp
from jax.experimental import pallas as pl
from jax.experimental.pallas import tpu as pltpu

_B = 16
_NH = 8
_D = 128
_H = 64
_NC = 2
_BLK = 4096


def _mil_kernel(ends_ref, flat_ref, w1_ref, b1_ref, w2e_ref,
                b2e_ref, wc_ref, bc_ref, out_ref, acc_ref, accw_ref,
                iota_ref):
    i = pl.program_id(0)
    nsteps = pl.num_programs(0)

    @pl.when(i == 0)
    def _init():
        acc_ref[...] = jnp.zeros_like(acc_ref)
        accw_ref[...] = jnp.zeros_like(accw_ref)
        iota_ref[...] = jax.lax.broadcasted_iota(
            jnp.int32, (_BLK, _B * _NH), 0)

    x = flat_ref[...]
    h = jnp.tanh(
        jnp.dot(x, w1_ref[...], preferred_element_type=jnp.float32)
        + b1_ref[...])
    pre = (jnp.dot(h, w2e_ref[...], preferred_element_type=jnp.float32)
           + b2e_ref[...])
    att_big = (jnp.maximum(pre, 0.0)
               + jnp.log1p(jnp.exp(-jnp.abs(pre))))
    thr = ends_ref[...] - i * _BLK
    m = jnp.where(iota_ref[...] < thr, att_big, 0.0)
    acc_ref[...] += jax.lax.dot_general(
        m, x, (((0,), (0,)), ((), ())), preferred_element_type=jnp.float32)
    accw_ref[...] += jnp.sum(m, axis=0, keepdims=True)

    @pl.when(i == nsteps - 1)
    def _finish():
        eye = (jax.lax.broadcasted_iota(jnp.int32, (_B * _NH, _B * _NH), 0)
               == jax.lax.broadcasted_iota(
                   jnp.int32, (_B * _NH, _B * _NH), 1)).astype(jnp.float32)
        wpre = jax.lax.dot_general(
            eye, accw_ref[...], (((1,), (1,)), ((), ())),
            preferred_element_type=jnp.float32)
        rid = jax.lax.broadcasted_iota(jnp.int32, (_B * _NH, 1), 0)
        first = (rid % _B) == 0
        pacc = acc_ref[...]
        prev_acc = jnp.where(
            first, 0.0,
            jnp.concatenate([jnp.zeros((1, _D), jnp.float32), pacc[:-1, :]],
                            axis=0))
        prev_w = jnp.where(
            first, 0.0,
            jnp.concatenate([jnp.zeros((1, 1), jnp.float32), wpre[:-1, :]],
                            axis=0))
        wsum = wpre - prev_w
        denom = jnp.where(wsum == 0.0, 1.0, wsum)
        norm = (pacc - prev_acc) / denom
        logits = jnp.zeros((_B, _NC), dtype=jnp.float32) + bc_ref[...]
        for j in range(_NH):
            logits = logits + jnp.dot(
                norm[j * _B:(j + 1) * _B, :],
                wc_ref[j * _D:(j + 1) * _D, :],
                preferred_element_type=jnp.float32)
        out_ref[...] = logits


@jax.jit
def kernel(flat, cu_seqlens, W1, b1, W2, b2, Wc, bc):
    total = flat.shape[0]
    assert total % _BLK == 0
    nsteps = total // _BLK
    ends = jnp.tile(cu_seqlens[1:_B + 1], (_NH,)).reshape(1, _B * _NH)
    w2e = jnp.repeat(W2, _B, axis=1)
    b2e = jnp.repeat(b2, _B).reshape(1, _B * _NH)

    return pl.pallas_call(
        _mil_kernel,
        grid=(nsteps,),
        in_specs=[
            pl.BlockSpec((1, _B * _NH), lambda i: (0, 0)),
            pl.BlockSpec((_BLK, _D), lambda i: (i, 0)),
            pl.BlockSpec((_D, _H), lambda i: (0, 0)),
            pl.BlockSpec((1, _H), lambda i: (0, 0)),
            pl.BlockSpec((_H, _B * _NH), lambda i: (0, 0)),
            pl.BlockSpec((1, _B * _NH), lambda i: (0, 0)),
            pl.BlockSpec((_NH * _D, _NC), lambda i: (0, 0)),
            pl.BlockSpec((1, _NC), lambda i: (0, 0)),
        ],
        out_specs=pl.BlockSpec((_B, _NC), lambda i: (0, 0)),
        out_shape=jax.ShapeDtypeStruct((_B, _NC), jnp.float32),
        scratch_shapes=[
            pltpu.VMEM((_B * _NH, _D), jnp.float32),
            pltpu.VMEM((1, _B * _NH), jnp.float32),
            pltpu.VMEM((_BLK, _B * _NH), jnp.int32),
        ],
        compiler_params=pltpu.CompilerParams(
            dimension_semantics=("arbitrary",)),
    )(ends, flat, W1, b1.reshape(1, _H), w2e, b2e,
      Wc, bc.reshape(1, _NC))


# speedup vs baseline: 99.1091x; 1.0924x over previous
import functools

import jax
import jax.numpy as jnp
from jax.experimental import pallas as pl
from jax.experimental.pallas import tpu as pltpu

_B = 16
_NH = 8
_D = 128
_H = 64
_NC = 2
_BLK = 4096


def _mil_kernel(cu_ref, flat_ref, w1_ref, b1_ref, w2_ref, b2_ref, wc_ref,
                bc_ref, out_ref, acc_ref, accw_ref, iota_ref, w2e_ref,
                b2e_ref, ends_ref):
    i = pl.program_id(0)
    nsteps = pl.num_programs(0)

    @pl.when(i == 0)
    def _init():
        acc_ref[...] = jnp.zeros_like(acc_ref)
        accw_ref[...] = jnp.zeros_like(accw_ref)
        iota_ref[...] = jax.lax.broadcasted_iota(
            jnp.int32, (_BLK, _B * _NH), 0)
        cut = cu_ref[0:1, 1:_B + 1]
        ends_ref[...] = jnp.concatenate([cut] * _NH, axis=1)
        r8 = (jax.lax.broadcasted_iota(jnp.int32, (_NH, _B * _NH), 0)
              == jax.lax.broadcasted_iota(
                  jnp.int32, (_NH, _B * _NH), 1) // _B).astype(jnp.float32)
        w2e_ref[...] = jnp.dot(w2_ref[...], r8,
                               preferred_element_type=jnp.float32)
        b2e_ref[...] = jnp.dot(b2_ref[...], r8,
                               preferred_element_type=jnp.float32)

    x = flat_ref[...]
    h = jnp.tanh(
        jnp.dot(x, w1_ref[...], preferred_element_type=jnp.float32)
        + b1_ref[...])
    pre = (jnp.dot(h, w2e_ref[...], preferred_element_type=jnp.float32)
           + b2e_ref[...])
    att_big = (jnp.maximum(pre, 0.0)
               + jnp.log1p(jnp.exp(-jnp.abs(pre))))
    thr = ends_ref[...] - i * _BLK
    m = jnp.where(iota_ref[...] < thr, att_big, 0.0)
    acc_ref[...] += jax.lax.dot_general(
        m, x, (((0,), (0,)), ((), ())), preferred_element_type=jnp.float32)
    accw_ref[...] += jnp.sum(m, axis=0, keepdims=True)

    @pl.when(i == nsteps - 1)
    def _finish():
        eye = (jax.lax.broadcasted_iota(jnp.int32, (_B * _NH, _B * _NH), 0)
               == jax.lax.broadcasted_iota(
                   jnp.int32, (_B * _NH, _B * _NH), 1)).astype(jnp.float32)
        wpre = jax.lax.dot_general(
            eye, accw_ref[...], (((1,), (1,)), ((), ())),
            preferred_element_type=jnp.float32)
        rid = jax.lax.broadcasted_iota(jnp.int32, (_B * _NH, 1), 0)
        first = (rid % _B) == 0
        pacc = acc_ref[...]
        prev_acc = jnp.where(
            first, 0.0,
            jnp.concatenate([jnp.zeros((1, _D), jnp.float32), pacc[:-1, :]],
                            axis=0))
        prev_w = jnp.where(
            first, 0.0,
            jnp.concatenate([jnp.zeros((1, 1), jnp.float32), wpre[:-1, :]],
                            axis=0))
        wsum = wpre - prev_w
        denom = jnp.where(wsum == 0.0, 1.0, wsum)
        norm = (pacc - prev_acc) / denom
        logits = jnp.zeros((_B, _NC), dtype=jnp.float32) + bc_ref[...]
        for j in range(_NH):
            logits = logits + jnp.dot(
                norm[j * _B:(j + 1) * _B, :],
                wc_ref[j * _D:(j + 1) * _D, :],
                preferred_element_type=jnp.float32)
        out_ref[...] = logits


@jax.jit
def kernel(flat, cu_seqlens, W1, b1, W2, b2, Wc, bc):
    total = flat.shape[0]
    assert total % _BLK == 0
    nsteps = total // _BLK

    return pl.pallas_call(
        _mil_kernel,
        grid=(nsteps,),
        in_specs=[
            pl.BlockSpec((1, _B + 1), lambda i: (0, 0)),
            pl.BlockSpec((_BLK, _D), lambda i: (i, 0)),
            pl.BlockSpec((_D, _H), lambda i: (0, 0)),
            pl.BlockSpec((1, _H), lambda i: (0, 0)),
            pl.BlockSpec((_H, _NH), lambda i: (0, 0)),
            pl.BlockSpec((1, _NH), lambda i: (0, 0)),
            pl.BlockSpec((_NH * _D, _NC), lambda i: (0, 0)),
            pl.BlockSpec((1, _NC), lambda i: (0, 0)),
        ],
        out_specs=pl.BlockSpec((_B, _NC), lambda i: (0, 0)),
        out_shape=jax.ShapeDtypeStruct((_B, _NC), jnp.float32),
        scratch_shapes=[
            pltpu.VMEM((_B * _NH, _D), jnp.float32),
            pltpu.VMEM((1, _B * _NH), jnp.float32),
            pltpu.VMEM((_BLK, _B * _NH), jnp.int32),
            pltpu.VMEM((_H, _B * _NH), jnp.float32),
            pltpu.VMEM((1, _B * _NH), jnp.float32),
            pltpu.VMEM((1, _B * _NH), jnp.int32),
        ],
        compiler_params=pltpu.CompilerParams(
            dimension_semantics=("arbitrary",)),
    )(cu_seqlens.reshape(1, _B + 1), flat, W1, b1.reshape(1, _H), W2,
      b2.reshape(1, _NH), Wc, bc.reshape(1, _NC))
